# baseline (speedup 1.0000x reference)
import functools

import jax
import jax.numpy as jnp
from jax import lax
from jax.experimental import pallas as pl
from jax.experimental.pallas import tpu as pltpu
from jax.experimental.pallas import tpu_sc as plsc

NUM_TYPES = 3
VOCAB = 100000
D = 64
B, L, F = 1024, 50, 4
N = B * L

NC, NS = 2, 16
NW = NC * NS
PER_W = N // NW
C = 80
NCHUNK = PER_W // C
LANES = 16


def _sc_embed(ids_il, types_il, tag_flat, cat_flat, table, tag_table,
              cat_table):
    mesh = plsc.VectorSubcoreMesh(
        core_axis_name="c", subcore_axis_name="s", num_cores=NC, num_subcores=NS
    )

    @functools.partial(
        pl.kernel,
        out_type=jax.ShapeDtypeStruct((N, D), jnp.float32),
        mesh=mesh,
        compiler_params=pltpu.CompilerParams(use_tc_tiling_on_sc=False),
        scratch_types=dict(
            ids_v=pltpu.VMEM((F * PER_W,), jnp.int32),
            types_v=pltpu.VMEM((F * PER_W,), jnp.int32),
            idx_v=pltpu.VMEM((F * PER_W,), jnp.int32),
            tag_v=pltpu.VMEM((PER_W,), jnp.int32),
            cat_v=pltpu.VMEM((PER_W,), jnp.int32),
            gm=pltpu.VMEM((2, F * C, D), jnp.float32),
            gt=pltpu.VMEM((2, C, D), jnp.float32),
            gc=pltpu.VMEM((2, C, D), jnp.float32),
            ob=pltpu.VMEM((2, C, D), jnp.float32),
            isem=pltpu.SemaphoreType.DMA,
            gsem0=pltpu.SemaphoreType.DMA,
            gsem1=pltpu.SemaphoreType.DMA,
            osem0=pltpu.SemaphoreType.DMA,
            osem1=pltpu.SemaphoreType.DMA,
        ),
    )
    def body(ids_hbm, types_hbm, tag_hbm, cat_hbm, table_hbm, tagt_hbm,
             catt_hbm, out_hbm, *, ids_v, types_v, idx_v, tag_v, cat_v, gm,
             gt, gc, ob, isem, gsem0, gsem1, osem0, osem1):
        wid = lax.axis_index("s") * NC + lax.axis_index("c")
        base0 = wid * PER_W
        gsems = (gsem0, gsem1)
        osems = (osem0, osem1)

        stage = [
            pltpu.async_copy(ids_hbm.at[pl.ds(base0 * F, PER_W * F)], ids_v,
                             isem),
            pltpu.async_copy(types_hbm.at[pl.ds(base0 * F, PER_W * F)],
                             types_v, isem),
            pltpu.async_copy(tag_hbm.at[pl.ds(base0, PER_W)], tag_v, isem),
            pltpu.async_copy(cat_hbm.at[pl.ds(base0, PER_W)], cat_v, isem),
        ]
        for h in stage:
            h.wait()

        def ix_body(i, carry):
            s = pl.ds(i * LANES, LANES)
            idx_v[s] = types_v[s] * VOCAB + ids_v[s]
            return carry
        lax.fori_loop(0, (F * PER_W) // LANES, ix_body, 0)

        def fire(k, b):
            cs = pl.ds(k * C, C)
            hs = []
            for q in range(F):
                hs.append(pltpu.async_copy(
                    table_hbm.at[idx_v.at[pl.ds((k * F + q) * C, C)]],
                    gm.at[b, pl.ds(q * C, C)], gsems[b]))
            hs.append(pltpu.async_copy(tagt_hbm.at[tag_v.at[cs]], gt.at[b],
                                       gsems[b]))
            hs.append(pltpu.async_copy(catt_hbm.at[cat_v.at[cs]], gc.at[b],
                                       gsems[b]))
            return hs

        def compute(b):
            def row_body(c, carry):
                r = c * F
                for j in range(D // LANES):
                    s = pl.ds(j * LANES, LANES)
                    acc = gm[b, r, s] + gm[b, r + 1, s]
                    acc = acc + gm[b, r + 2, s]
                    acc = acc + gm[b, r + 3, s]
                    acc = acc + gt[b, c, s]
                    ob[b, c, s] = acc + gc[b, c, s]
                return carry
            lax.fori_loop(0, C, row_body, 0)

        ghandles = [None, None]
        ohandles = [None, None]
        ghandles[0] = fire(0, 0)
        for k in range(NCHUNK):
            b = k & 1
            if k + 1 < NCHUNK:
                ghandles[1 - b] = fire(k + 1, 1 - b)
            for h in ghandles[b]:
                h.wait()
            if ohandles[b] is not None:
                ohandles[b].wait()
            compute(b)
            ohandles[b] = pltpu.async_copy(
                ob.at[b], out_hbm.at[pl.ds(base0 + k * C, C)], osems[b])
        for h in ohandles:
            if h is not None:
                h.wait()

    return body(ids_il, types_il, tag_flat, cat_flat, table, tag_table,
                cat_table)


def kernel(input_ids, type_ids, feat_tag_ids, feat_cat_ids, type_tables,
           tag_table, cat_table):
    ids_il = input_ids.reshape(N * F)
    types_il = type_ids.reshape(N * F)
    tag_flat = feat_tag_ids.reshape(N)
    cat_flat = feat_cat_ids.reshape(N)
    table = type_tables.reshape(NUM_TYPES * VOCAB, D)
    out = _sc_embed(ids_il, types_il, tag_flat, cat_flat, table, tag_table,
                    cat_table)
    return out.reshape(B, L, D)


# speedup vs baseline: 1.2817x; 1.2817x over previous
import functools

import jax
import jax.numpy as jnp
from jax import lax
from jax.experimental import pallas as pl
from jax.experimental.pallas import tpu as pltpu
from jax.experimental.pallas import tpu_sc as plsc

NUM_TYPES = 3
VOCAB = 100000
FEAT_VOCAB = 1000
D = 64
B, L, F = 1024, 50, 4
N = B * L

NC, NS = 2, 16
NW = NC * NS
PER_W = N // NW
C = 80
NCHUNK = PER_W // C
LANES = 16


def _sc_embed(idx_main, idx_tc, table, tc_table):
    mesh = plsc.VectorSubcoreMesh(
        core_axis_name="c", subcore_axis_name="s", num_cores=NC, num_subcores=NS
    )

    @functools.partial(
        pl.kernel,
        out_type=jax.ShapeDtypeStruct((N, D), jnp.float32),
        mesh=mesh,
        compiler_params=pltpu.CompilerParams(use_tc_tiling_on_sc=False),
        scratch_types=dict(
            idx_v=pltpu.VMEM((F * PER_W,), jnp.int32),
            tag_v=pltpu.VMEM((PER_W,), jnp.int32),
            cat_v=pltpu.VMEM((PER_W,), jnp.int32),
            g=pltpu.VMEM((2, 6, C, D), jnp.float32),
            ob=pltpu.VMEM((2, C, D), jnp.float32),
            isem=pltpu.SemaphoreType.DMA,
            gsem0=pltpu.SemaphoreType.DMA,
            gsem1=pltpu.SemaphoreType.DMA,
            osem0=pltpu.SemaphoreType.DMA,
            osem1=pltpu.SemaphoreType.DMA,
        ),
    )
    def body(idx_hbm, idxtc_hbm, table_hbm, tct_hbm, out_hbm, *, idx_v,
             tag_v, cat_v, g, ob, isem, gsem0, gsem1, osem0, osem1):
        wid = lax.axis_index("s") * NC + lax.axis_index("c")
        base0 = wid * PER_W
        gsems = (gsem0, gsem1)
        osems = (osem0, osem1)

        stage = [
            pltpu.async_copy(idxtc_hbm.at[pl.ds(base0, PER_W)], tag_v, isem),
            pltpu.async_copy(idxtc_hbm.at[pl.ds(N + base0, PER_W)], cat_v,
                             isem),
        ]
        for f in range(F):
            stage.append(pltpu.async_copy(
                idx_hbm.at[pl.ds(f * N + base0, PER_W)],
                idx_v.at[pl.ds(f * PER_W, PER_W)], isem))
        for h in stage:
            h.wait()

        def fire(k, b):
            cs = pl.ds(k * C, C)
            hs = []
            for f in range(F):
                hs.append(pltpu.async_copy(
                    table_hbm.at[idx_v.at[pl.ds(f * PER_W + k * C, C)]],
                    g.at[b, f], gsems[b]))
            hs.append(pltpu.async_copy(tct_hbm.at[tag_v.at[cs]], g.at[b, 4],
                                       gsems[b]))
            hs.append(pltpu.async_copy(tct_hbm.at[cat_v.at[cs]], g.at[b, 5],
                                       gsems[b]))
            return hs

        def compute(b):
            def row_body(c, carry):
                for j in range(D // LANES):
                    s = pl.ds(j * LANES, LANES)
                    acc = g[b, 0, c, s] + g[b, 1, c, s]
                    acc = acc + g[b, 2, c, s]
                    acc = acc + g[b, 3, c, s]
                    acc = acc + g[b, 4, c, s]
                    ob[b, c, s] = acc + g[b, 5, c, s]
                return carry
            lax.fori_loop(0, C, row_body, 0)

        ghandles = [None, None]
        ohandles = [None, None]
        ghandles[0] = fire(0, 0)
        for k in range(NCHUNK):
            b = k & 1
            if k + 1 < NCHUNK:
                ghandles[1 - b] = fire(k + 1, 1 - b)
            for h in ghandles[b]:
                h.wait()
            if ohandles[b] is not None:
                ohandles[b].wait()
            compute(b)
            ohandles[b] = pltpu.async_copy(
                ob.at[b], out_hbm.at[pl.ds(base0 + k * C, C)], osems[b])
        for h in ohandles:
            if h is not None:
                h.wait()

    return body(idx_main, idx_tc, table, tc_table)


def kernel(input_ids, type_ids, feat_tag_ids, feat_cat_ids, type_tables,
           tag_table, cat_table):
    idx_main = (type_ids * VOCAB + input_ids).reshape(N, F).T.reshape(F * N)
    idx_tc = jnp.concatenate(
        [feat_tag_ids.reshape(N), feat_cat_ids.reshape(N) + FEAT_VOCAB])
    table = type_tables.reshape(NUM_TYPES * VOCAB, D)
    tc_table = jnp.concatenate([tag_table, cat_table], axis=0)
    out = _sc_embed(idx_main, idx_tc, table, tc_table)
    return out.reshape(B, L, D)
